# baseline (speedup 1.0000x reference)
import functools

import jax
import jax.numpy as jnp
from jax.experimental import pallas as pl
from jax.experimental.pallas import tpu as pltpu

NUM_EXPERTS = 64
TOP_K = 8
T_BLK = 2048


def _gating_kernel(x_ref, w_ref, b_ref, sw_ref, idx_ref, acc_ref, *, nt, nb, seq_len):
    t = pl.program_id(0)
    b = pl.program_id(1)

    @pl.when((t == 0) & (b == 0))
    def _init():
        acc_ref[...] = jnp.zeros_like(acc_ref)

    acc_ref[pl.ds(b, 1), :] += jnp.sum(x_ref[...], axis=1)

    @pl.when((t == nt - 1) & (b == nb - 1))
    def _finish():
        pooled = acc_ref[...] * (1.0 / seq_len)
        logits = jax.lax.dot_general(
            pooled.astype(jnp.bfloat16), w_ref[...].astype(jnp.bfloat16),
            (((1,), (1,)), ((), ())),
            preferred_element_type=jnp.float32,
        ) + b_ref[...]

        e_iota = jax.lax.broadcasted_iota(jnp.int32, logits.shape, 1)
        vals = logits
        top_vals = []
        top_idx = []
        for _ in range(TOP_K):
            m = jnp.max(vals, axis=1, keepdims=True)
            i = jnp.min(jnp.where(vals == m, e_iota, NUM_EXPERTS),
                        axis=1, keepdims=True)
            top_vals.append(m)
            top_idx.append(i)
            vals = jnp.where(e_iota == i, -jnp.inf, vals)

        tv = jnp.concatenate(top_vals, axis=1)
        ex = jnp.exp(tv - tv[:, :1])
        probs = ex / jnp.sum(ex, axis=1, keepdims=True)

        sparse = jnp.zeros_like(logits)
        for k in range(TOP_K):
            sparse += jnp.where(e_iota == top_idx[k], probs[:, k:k + 1], 0.0)

        sw_ref[...] = sparse
        idx_ref[...] = jnp.concatenate(top_idx, axis=1)


@jax.jit
def kernel(x, W, b):
    B, T, D = x.shape
    nt = T // T_BLK
    sw, idx = pl.pallas_call(
        functools.partial(_gating_kernel, nt=nt, nb=B, seq_len=T),
        grid=(nt, B),
        in_specs=[
            pl.BlockSpec((1, T_BLK, D), lambda t, b: (b, t, 0)),
            pl.BlockSpec((NUM_EXPERTS, D), lambda t, b: (0, 0)),
            pl.BlockSpec((1, NUM_EXPERTS), lambda t, b: (0, 0)),
        ],
        out_specs=[
            pl.BlockSpec((B, NUM_EXPERTS), lambda t, b: (0, 0)),
            pl.BlockSpec((B, TOP_K), lambda t, b: (0, 0)),
        ],
        out_shape=[
            jax.ShapeDtypeStruct((B, NUM_EXPERTS), jnp.float32),
            jax.ShapeDtypeStruct((B, TOP_K), jnp.int32),
        ],
        scratch_shapes=[pltpu.VMEM((B, D), jnp.float32)],
    )(x, W, b.reshape(1, NUM_EXPERTS))
    return (sw, idx)


# speedup vs baseline: 1.0091x; 1.0091x over previous
import functools

import jax
import jax.numpy as jnp
from jax.experimental import pallas as pl
from jax.experimental.pallas import tpu as pltpu

NUM_EXPERTS = 64
TOP_K = 8
T_BLK = 1024


def _gating_kernel(x_ref, w_ref, b_ref, sw_ref, idx_ref, acc_ref, lg_ref,
                   *, nt, nb, seq_len):
    b = pl.program_id(0)
    t = pl.program_id(1)

    @pl.when(t == 0)
    def _init():
        acc_ref[...] = jnp.zeros_like(acc_ref)

    acc_ref[...] += jnp.sum(x_ref[...], axis=1)

    @pl.when(t == nt - 1)
    def _batch_logits():
        pooled = acc_ref[...] * (1.0 / seq_len)
        lg_ref[pl.ds(b, 1), :] = jax.lax.dot_general(
            pooled.astype(jnp.bfloat16), w_ref[...].astype(jnp.bfloat16),
            (((1,), (1,)), ((), ())),
            preferred_element_type=jnp.float32,
        ) + b_ref[...]

    @pl.when((b == nb - 1) & (t == nt - 1))
    def _finish():
        logits = lg_ref[...]
        e_iota = jax.lax.broadcasted_iota(jnp.int32, logits.shape, 1)
        vals = logits
        top_vals = []
        top_idx = []
        for _ in range(TOP_K):
            m = jnp.max(vals, axis=1, keepdims=True)
            i = jnp.min(jnp.where(vals == m, e_iota, NUM_EXPERTS),
                        axis=1, keepdims=True)
            top_vals.append(m)
            top_idx.append(i)
            vals = jnp.where(e_iota == i, -jnp.inf, vals)

        tv = jnp.concatenate(top_vals, axis=1)
        ex = jnp.exp(tv - tv[:, :1])
        probs = ex / jnp.sum(ex, axis=1, keepdims=True)

        sparse = jnp.zeros_like(logits)
        for k in range(TOP_K):
            sparse += jnp.where(e_iota == top_idx[k], probs[:, k:k + 1], 0.0)

        sw_ref[...] = sparse
        idx_ref[...] = jnp.concatenate(top_idx, axis=1)


@jax.jit
def kernel(x, W, b):
    B, T, D = x.shape
    nt = T // T_BLK
    sw, idx = pl.pallas_call(
        functools.partial(_gating_kernel, nt=nt, nb=B, seq_len=T),
        grid=(B, nt),
        in_specs=[
            pl.BlockSpec((1, T_BLK, D), lambda b, t: (b, t, 0)),
            pl.BlockSpec((NUM_EXPERTS, D), lambda b, t: (0, 0)),
            pl.BlockSpec((1, NUM_EXPERTS), lambda b, t: (0, 0)),
        ],
        out_specs=[
            pl.BlockSpec((B, NUM_EXPERTS), lambda b, t: (0, 0)),
            pl.BlockSpec((B, TOP_K), lambda b, t: (0, 0)),
        ],
        out_shape=[
            jax.ShapeDtypeStruct((B, NUM_EXPERTS), jnp.float32),
            jax.ShapeDtypeStruct((B, TOP_K), jnp.int32),
        ],
        scratch_shapes=[
            pltpu.VMEM((1, D), jnp.float32),
            pltpu.VMEM((B, NUM_EXPERTS), jnp.float32),
        ],
    )(x, W, b.reshape(1, NUM_EXPERTS))
    return (sw, idx)
